# baseline (speedup 1.0000x reference)
import jax
import jax.numpy as jnp
from jax.experimental import pallas as pl

N_USERS = 25000
N_NODES = 50000
N_LAYERS = 3
LAM = 0.001
BATCH = 4096


def _loss_body(pos_s_ref, neg_s_ref, reg_ref, out_ref):
    x = neg_s_ref[...] - pos_s_ref[...]
    sp = jnp.maximum(x, 0.0) + jnp.log1p(jnp.exp(-jnp.abs(x)))
    loss = jnp.mean(sp) + LAM * 0.5 * reg_ref[0] / float(BATCH)
    out_ref[...] = jnp.full((8, 128), loss, dtype=jnp.float32)


def kernel(embedding_table, edge_index, users, pos, neg):
    ones = jnp.ones((edge_index.shape[1],), dtype=jnp.float32)
    deg = jax.ops.segment_sum(ones, edge_index[1], num_segments=N_NODES)
    sqrt_deg = jnp.where(deg > 0, jax.lax.rsqrt(jnp.maximum(deg, 1.0)), 0.0)[:, None]
    ego = embedding_table
    acc = ego
    for _ in range(N_LAYERS):
        node = ego * sqrt_deg
        msgs = jnp.take(node, edge_index[0], axis=0)
        ego = jax.ops.segment_sum(msgs, edge_index[1], num_segments=N_NODES) * sqrt_deg
        acc = acc + ego
    propagated = acc / float(N_LAYERS + 1)

    users_ego = jnp.take(embedding_table, users, axis=0)
    pos_ego = jnp.take(embedding_table, pos + N_USERS, axis=0)
    neg_ego = jnp.take(embedding_table, neg + N_USERS, axis=0)
    reg = jnp.sum(users_ego**2) + jnp.sum(pos_ego**2) + jnp.sum(neg_ego**2)

    users_emb = jnp.take(propagated, users, axis=0)
    pos_emb = jnp.take(propagated, pos + N_USERS, axis=0)
    neg_emb = jnp.take(propagated, neg + N_USERS, axis=0)
    pos_scores = jnp.sum(users_emb * pos_emb, axis=1)
    neg_scores = jnp.sum(users_emb * neg_emb, axis=1)

    out = pl.pallas_call(
        _loss_body,
        out_shape=jax.ShapeDtypeStruct((8, 128), jnp.float32),
    )(pos_scores, neg_scores, reg[None])
    return out[0, 0]


# speedup vs baseline: 4.9388x; 4.9388x over previous
import functools

import jax
import jax.numpy as jnp
from jax import lax
from jax.experimental import pallas as pl
from jax.experimental.pallas import tpu as pltpu
from jax.experimental.pallas import tpu_sc as plsc

N_USERS = 25000
N_NODES = 50000
N_LAYERS = 3
LAM = 0.001
BATCH = 4096

NC, NS = 2, 16
NW = NC * NS
NPAD = 50048
ROWS_PER_TILE = NPAD // NS
HALF = 32
N_EDGES = 800000
EPAD = 819200
CHUNK = 128
LAYER_CHUNKS = EPAD // NS // CHUNK
DEG_EDGES = EPAD // NW
DEGN = 51200
TRASH = N_NODES

_MESH = plsc.VectorSubcoreMesh(core_axis_name="c", subcore_axis_name="s")
_SC_PARAMS = pltpu.CompilerParams(use_tc_tiling_on_sc=False)


DEG_CHUNKS = EPAD // NW // CHUNK


def _deg_body(dst_hbm, ones_hbm, zeros_hbm, deg_out,
              deg_sh, dst_v, ones_v):
    c = lax.axis_index("c")
    s = lax.axis_index("s")
    t = c * NS + s
    row0 = s * ROWS_PER_TILE
    pltpu.sync_copy(zeros_hbm, deg_sh.at[pl.ds(row0, ROWS_PER_TILE)])
    pltpu.sync_copy(ones_hbm, ones_v)
    pltpu.sync_copy(dst_hbm.at[t], dst_v)
    plsc.subcore_barrier()

    def chunk_body(j, carry):
        pltpu.sync_copy(ones_v, deg_sh.at[dst_v.at[j]], add=True)
        return carry

    lax.fori_loop(0, DEG_CHUNKS, chunk_body, 0)
    plsc.subcore_barrier()
    pltpu.sync_copy(deg_sh.at[pl.ds(row0, ROWS_PER_TILE)],
                    deg_out.at[c, pl.ds(row0, ROWS_PER_TILE)])


_deg_call = pl.kernel(
    _deg_body,
    compiler_params=_SC_PARAMS,
    out_type=jax.ShapeDtypeStruct((NC, NPAD, 16), jnp.float32),
    mesh=_MESH,
    scratch_types=[
        pltpu.VMEM_SHARED((NPAD, 16), jnp.float32),
        pltpu.VMEM((DEG_CHUNKS, CHUNK), jnp.int32),
        pltpu.VMEM((CHUNK, 16), jnp.float32),
    ],
)


_IDX_BLK = 50
_N_BLK = LAYER_CHUNKS // _IDX_BLK


def _layer_body(node2, srcs, dsts, zeros_hbm, y_out,
                acc_sh, src_v, dst_v, msg0, msg1, sem0, sem1):
    c = lax.axis_index("c")
    s = lax.axis_index("s")
    row0 = s * ROWS_PER_TILE
    pltpu.sync_copy(zeros_hbm, acc_sh.at[pl.ds(row0, ROWS_PER_TILE)])
    plsc.subcore_barrier()

    def outer_body(b, carry):
        pltpu.sync_copy(srcs.at[c, s, pl.ds(b * _IDX_BLK, _IDX_BLK)], src_v)
        pltpu.sync_copy(dsts.at[s, pl.ds(b * _IDX_BLK, _IDX_BLK)], dst_v)
        pltpu.async_copy(node2.at[src_v.at[0]], msg0, sem0)

        def chunk_body(g, carry2):
            j0 = 2 * g
            pltpu.make_async_copy(node2.at[src_v.at[j0]], msg0, sem0).wait()
            pltpu.async_copy(node2.at[src_v.at[j0 + 1]], msg1, sem1)
            pltpu.sync_copy(msg0, acc_sh.at[dst_v.at[j0]], add=True)
            pltpu.make_async_copy(node2.at[src_v.at[j0 + 1]], msg1,
                                  sem1).wait()

            @pl.when(g < _IDX_BLK // 2 - 1)
            def _():
                pltpu.async_copy(node2.at[src_v.at[j0 + 2]], msg0, sem0)

            pltpu.sync_copy(msg1, acc_sh.at[dst_v.at[j0 + 1]], add=True)
            return carry2

        lax.fori_loop(0, _IDX_BLK // 2, chunk_body, 0)
        return carry

    lax.fori_loop(0, _N_BLK, outer_body, 0)
    plsc.subcore_barrier()
    pltpu.sync_copy(acc_sh.at[pl.ds(row0, ROWS_PER_TILE)],
                    y_out.at[c, pl.ds(row0, ROWS_PER_TILE)])


_layer_call = pl.kernel(
    _layer_body,
    compiler_params=_SC_PARAMS,
    out_type=jax.ShapeDtypeStruct((NC, NPAD, HALF), jnp.float32),
    mesh=_MESH,
    scratch_types=[
        pltpu.VMEM_SHARED((NPAD, HALF), jnp.float32),
        pltpu.VMEM((_IDX_BLK, CHUNK), jnp.int32),
        pltpu.VMEM((_IDX_BLK, CHUNK), jnp.int32),
        pltpu.VMEM((CHUNK, HALF), jnp.float32),
        pltpu.VMEM((CHUNK, HALF), jnp.float32),
        pltpu.SemaphoreType.DMA,
        pltpu.SemaphoreType.DMA,
    ],
)


def _gather_body(table, prop2, ie_hbm, ip_hbm, ego_out, prop_out,
                 ib, rb64, rb32, sem):
    c = lax.axis_index("c")
    s = lax.axis_index("s")
    t = c * NS + s
    for k in range(3):
        pltpu.sync_copy(ie_hbm.at[k, t], ib)
        pltpu.async_copy(table.at[ib], rb64, sem).wait()
        pltpu.sync_copy(rb64, ego_out.at[k, pl.ds(t * CHUNK, CHUNK)])
    for k in range(6):
        pltpu.sync_copy(ip_hbm.at[k, t], ib)
        pltpu.async_copy(prop2.at[ib], rb32, sem).wait()
        pltpu.sync_copy(rb32, prop_out.at[k, pl.ds(t * CHUNK, CHUNK)])


_gather_call = pl.kernel(
    _gather_body,
    compiler_params=_SC_PARAMS,
    out_type=(
        jax.ShapeDtypeStruct((3, BATCH, 64), jnp.float32),
        jax.ShapeDtypeStruct((6, BATCH, HALF), jnp.float32),
    ),
    mesh=_MESH,
    scratch_types=[
        pltpu.VMEM((CHUNK,), jnp.int32),
        pltpu.VMEM((CHUNK, 64), jnp.float32),
        pltpu.VMEM((CHUNK, HALF), jnp.float32),
        pltpu.SemaphoreType.DMA,
    ],
)


_SD_RB = 2176


def _sd_body(deg_ref, sd_ref):
    b = deg_ref[...]
    degsum = jnp.sum(b, axis=(0, 2)) * (1.0 / 16.0)
    sd = jnp.where(degsum > 0, lax.rsqrt(jnp.maximum(degsum, 1.0)), 0.0)
    sd_ref[...] = sd[:, None]


def _sd_call(deg_parts):
    return pl.pallas_call(
        _sd_body,
        grid=(NPAD // _SD_RB,),
        in_specs=[pl.BlockSpec((NC, _SD_RB, 16), lambda b: (0, b, 0))],
        out_specs=pl.BlockSpec((_SD_RB, 1), lambda b: (b, 0)),
        out_shape=jax.ShapeDtypeStruct((NPAD, 1), jnp.float32),
    )(deg_parts)


_PREP_RB = 2176


def _prep_body(tab_ref, sd_ref, node0_ref, tsplit_ref, sdb_ref, sd2b_ref):
    h = pl.program_id(1)
    sd = sd_ref[...]
    sdc = jnp.broadcast_to(sd, (_PREP_RB, HALF))
    tfull = tab_ref[...]
    tb = jnp.where(h == 0, tfull[:, :HALF], tfull[:, HALF:])
    node0_ref[...] = (tb * sdc)[None]
    tsplit_ref[...] = tb[None]
    sdb_ref[...] = sdc[None]
    sd2b_ref[...] = (sdc * sdc)[None]


def _prep_call(table, sd2d):
    shp = jax.ShapeDtypeStruct((NC, NPAD, HALF), jnp.float32)
    return pl.pallas_call(
        _prep_body,
        grid=(NPAD // _PREP_RB, NC),
        in_specs=[
            pl.BlockSpec((_PREP_RB, 64), lambda b, h: (b, 0)),
            pl.BlockSpec((_PREP_RB, 1), lambda b, h: (b, 0)),
        ],
        out_specs=[pl.BlockSpec((1, _PREP_RB, HALF), lambda b, h: (h, b, 0))] * 4,
        out_shape=(shp,) * 4,
    )(table, sd2d)


_EW_BLK = ROWS_PER_TILE


def _scale_body(y_ref, sd2b_ref, o_ref):
    o_ref[...] = y_ref[...] * sd2b_ref[...]


def _scale_call(y, sd2b):
    spec = pl.BlockSpec((NC, _EW_BLK, HALF), lambda b: (0, b, 0))
    return pl.pallas_call(
        _scale_body,
        grid=(NPAD // _EW_BLK,),
        in_specs=[spec, spec],
        out_specs=spec,
        out_shape=jax.ShapeDtypeStruct((NC, NPAD, HALF), jnp.float32),
    )(y, sd2b)


def _propfinal_body(t_ref, y1_ref, y2_ref, y3_ref, sdb_ref, o_ref):
    ysum = y1_ref[...] + y2_ref[...] + y3_ref[...]
    o_ref[...] = 0.25 * (t_ref[...] + ysum * sdb_ref[...])


def _propfinal_call(tsplit, y1, y2, y3, sdb):
    spec = pl.BlockSpec((NC, _EW_BLK, HALF), lambda b: (0, b, 0))
    return pl.pallas_call(
        _propfinal_body,
        grid=(NPAD // _EW_BLK,),
        in_specs=[spec] * 5,
        out_specs=spec,
        out_shape=jax.ShapeDtypeStruct((NC, NPAD, HALF), jnp.float32),
    )(tsplit, y1, y2, y3, sdb)


def _loss_body(ego_ref, pg_ref, out_ref):
    e = ego_ref[...]
    reg = jnp.sum(e * e)
    pg = pg_ref[...]
    pos_s = jnp.sum(pg[0] * pg[2] + pg[1] * pg[3], axis=1)
    neg_s = jnp.sum(pg[0] * pg[4] + pg[1] * pg[5], axis=1)
    x = neg_s - pos_s
    sp = jnp.maximum(x, 0.0) + jnp.log1p(jnp.exp(-jnp.abs(x)))
    loss = jnp.mean(sp) + LAM * 0.5 * reg / float(BATCH)
    out_ref[...] = jnp.full((8, 128), loss, jnp.float32)


def _loss_call(ego_g, prop_g):
    return pl.pallas_call(
        _loss_body,
        out_shape=jax.ShapeDtypeStruct((8, 128), jnp.float32),
    )(ego_g, prop_g)


def kernel(embedding_table, edge_index, users, pos, neg):
    src = edge_index[0]
    dst = edge_index[1]
    npad = EPAD - N_EDGES
    srcp = jnp.concatenate([src, jnp.zeros((npad,), jnp.int32)])
    dstp = jnp.concatenate([dst, jnp.full((npad,), TRASH, jnp.int32)])
    srcs = jnp.stack([srcp, srcp + NPAD]).reshape(NC, NS, LAYER_CHUNKS, CHUNK)
    dsts_layer = dstp.reshape(NS, LAYER_CHUNKS, CHUNK)
    dsts_deg = dstp.reshape(NW, DEG_CHUNKS, CHUNK)

    ones_rows = jnp.ones((CHUNK, 16), jnp.float32)
    zeros_deg = jnp.zeros((ROWS_PER_TILE, 16), jnp.float32)
    deg_parts = _deg_call(dsts_deg, ones_rows, zeros_deg)
    sd2d = _sd_call(deg_parts)
    node0, tsplit, sdb, sd2b = _prep_call(embedding_table, sd2d)

    zeros_tile = jnp.zeros((ROWS_PER_TILE, HALF), jnp.float32)
    node = node0
    ys = []
    for l in range(N_LAYERS):
        y = _layer_call(node.reshape(NC * NPAD, HALF), srcs, dsts_layer,
                        zeros_tile)
        ys.append(y)
        if l < N_LAYERS - 1:
            node = _scale_call(y, sd2b)
    prop = _propfinal_call(tsplit, ys[0], ys[1], ys[2], sdb)

    idx_ego = jnp.stack([users, pos + N_USERS, neg + N_USERS])
    idx_ego = idx_ego.reshape(3, NW, CHUNK)
    u0, p0, n0 = users, pos + N_USERS, neg + N_USERS
    idx_prop = jnp.stack([u0, u0 + NPAD, p0, p0 + NPAD, n0, n0 + NPAD])
    idx_prop = idx_prop.reshape(6, NW, CHUNK)
    ego_g, prop_g = _gather_call(embedding_table,
                                 prop.reshape(NC * NPAD, HALF),
                                 idx_ego, idx_prop)
    out = _loss_call(ego_g, prop_g)
    return out[0, 0]


# speedup vs baseline: 5.6201x; 1.1380x over previous
import functools

import jax
import jax.numpy as jnp
from jax import lax
from jax.experimental import pallas as pl
from jax.experimental.pallas import tpu as pltpu
from jax.experimental.pallas import tpu_sc as plsc

N_USERS = 25000
N_NODES = 50000
N_LAYERS = 3
LAM = 0.001
BATCH = 4096

NC, NS = 2, 16
NW = NC * NS
NPAD = 50048
ROWS_PER_TILE = NPAD // NS
HALF = 32
N_EDGES = 800000
EPAD = 819200
CHUNK = 256
LAYER_CHUNKS = EPAD // NS // CHUNK
DEG_EDGES = EPAD // NW
DEGN = 51200
TRASH = N_NODES

_MESH = plsc.VectorSubcoreMesh(core_axis_name="c", subcore_axis_name="s")
_SC_PARAMS = pltpu.CompilerParams(use_tc_tiling_on_sc=False)


DEG_CHUNK = 128
DEG_CHUNKS = EPAD // NW // DEG_CHUNK


def _deg_body(dst_hbm, ones_hbm, zeros_hbm, deg_out,
              deg_sh, dst_v, ones_v):
    c = lax.axis_index("c")
    s = lax.axis_index("s")
    t = c * NS + s
    row0 = s * ROWS_PER_TILE
    pltpu.sync_copy(zeros_hbm, deg_sh.at[pl.ds(row0, ROWS_PER_TILE)])
    pltpu.sync_copy(ones_hbm, ones_v)
    pltpu.sync_copy(dst_hbm.at[t], dst_v)
    plsc.subcore_barrier()

    def chunk_body(j, carry):
        pltpu.sync_copy(ones_v, deg_sh.at[dst_v.at[j]], add=True)
        return carry

    lax.fori_loop(0, DEG_CHUNKS, chunk_body, 0)
    plsc.subcore_barrier()
    pltpu.sync_copy(deg_sh.at[pl.ds(row0, ROWS_PER_TILE)],
                    deg_out.at[c, pl.ds(row0, ROWS_PER_TILE)])


_deg_call = pl.kernel(
    _deg_body,
    compiler_params=_SC_PARAMS,
    out_type=jax.ShapeDtypeStruct((NC, NPAD, 16), jnp.float32),
    mesh=_MESH,
    scratch_types=[
        pltpu.VMEM_SHARED((NPAD, 16), jnp.float32),
        pltpu.VMEM((DEG_CHUNKS, DEG_CHUNK), jnp.int32),
        pltpu.VMEM((DEG_CHUNK, 16), jnp.float32),
    ],
)


_IDX_BLK = 20
_N_BLK = LAYER_CHUNKS // _IDX_BLK


def _layer_body(node2, srcs, dsts, zeros_hbm, y_out,
                acc_sh, src_v, dst_v, msg0, msg1, sem0, sem1):
    c = lax.axis_index("c")
    s = lax.axis_index("s")
    row0 = s * ROWS_PER_TILE
    pltpu.sync_copy(zeros_hbm, acc_sh.at[pl.ds(row0, ROWS_PER_TILE)])
    plsc.subcore_barrier()

    def outer_body(b, carry):
        pltpu.sync_copy(srcs.at[c, s, pl.ds(b * _IDX_BLK, _IDX_BLK)], src_v)
        pltpu.sync_copy(dsts.at[s, pl.ds(b * _IDX_BLK, _IDX_BLK)], dst_v)
        pltpu.async_copy(node2.at[src_v.at[0]], msg0, sem0)

        def chunk_body(g, carry2):
            j0 = 2 * g
            pltpu.make_async_copy(node2.at[src_v.at[j0]], msg0, sem0).wait()
            pltpu.async_copy(node2.at[src_v.at[j0 + 1]], msg1, sem1)
            pltpu.sync_copy(msg0, acc_sh.at[dst_v.at[j0]], add=True)
            pltpu.make_async_copy(node2.at[src_v.at[j0 + 1]], msg1,
                                  sem1).wait()

            @pl.when(g < _IDX_BLK // 2 - 1)
            def _():
                pltpu.async_copy(node2.at[src_v.at[j0 + 2]], msg0, sem0)

            pltpu.sync_copy(msg1, acc_sh.at[dst_v.at[j0 + 1]], add=True)
            return carry2

        lax.fori_loop(0, _IDX_BLK // 2, chunk_body, 0)
        return carry

    lax.fori_loop(0, _N_BLK, outer_body, 0)
    plsc.subcore_barrier()
    pltpu.sync_copy(acc_sh.at[pl.ds(row0, ROWS_PER_TILE)],
                    y_out.at[c, pl.ds(row0, ROWS_PER_TILE)])


_layer_call = pl.kernel(
    _layer_body,
    compiler_params=_SC_PARAMS,
    out_type=jax.ShapeDtypeStruct((NC, NPAD, HALF), jnp.float32),
    mesh=_MESH,
    scratch_types=[
        pltpu.VMEM_SHARED((NPAD, HALF), jnp.float32),
        pltpu.VMEM((_IDX_BLK, CHUNK), jnp.int32),
        pltpu.VMEM((_IDX_BLK, CHUNK), jnp.int32),
        pltpu.VMEM((CHUNK, HALF), jnp.float32),
        pltpu.VMEM((CHUNK, HALF), jnp.float32),
        pltpu.SemaphoreType.DMA,
        pltpu.SemaphoreType.DMA,
    ],
)


def _gather_body(table, prop2, ie_hbm, ip_hbm, ego_out, prop_out,
                 ib, rb64, rb32, sem):
    c = lax.axis_index("c")
    s = lax.axis_index("s")
    t = c * NS + s
    for k in range(3):
        pltpu.sync_copy(ie_hbm.at[k, t], ib)
        pltpu.async_copy(table.at[ib], rb64, sem).wait()
        pltpu.sync_copy(rb64, ego_out.at[k, pl.ds(t * 128, 128)])
    for k in range(6):
        pltpu.sync_copy(ip_hbm.at[k, t], ib)
        pltpu.async_copy(prop2.at[ib], rb32, sem).wait()
        pltpu.sync_copy(rb32, prop_out.at[k, pl.ds(t * 128, 128)])


_gather_call = pl.kernel(
    _gather_body,
    compiler_params=_SC_PARAMS,
    out_type=(
        jax.ShapeDtypeStruct((3, BATCH, 64), jnp.float32),
        jax.ShapeDtypeStruct((6, BATCH, HALF), jnp.float32),
    ),
    mesh=_MESH,
    scratch_types=[
        pltpu.VMEM((128,), jnp.int32),
        pltpu.VMEM((128, 64), jnp.float32),
        pltpu.VMEM((128, HALF), jnp.float32),
        pltpu.SemaphoreType.DMA,
    ],
)


_SD_RB = 2176


def _sd_body(deg_ref, sd_ref):
    b = deg_ref[...]
    degsum = jnp.sum(b, axis=(0, 2)) * (1.0 / 16.0)
    sd = jnp.where(degsum > 0, lax.rsqrt(jnp.maximum(degsum, 1.0)), 0.0)
    sd_ref[...] = sd[:, None]


def _sd_call(deg_parts):
    return pl.pallas_call(
        _sd_body,
        grid=(NPAD // _SD_RB,),
        in_specs=[pl.BlockSpec((NC, _SD_RB, 16), lambda b: (0, b, 0))],
        out_specs=pl.BlockSpec((_SD_RB, 1), lambda b: (b, 0)),
        out_shape=jax.ShapeDtypeStruct((NPAD, 1), jnp.float32),
    )(deg_parts)


_PREP_RB = 2176


def _prep_body(tab_ref, sd_ref, node0_ref, tsplit_ref, sdb_ref, sd2b_ref):
    h = pl.program_id(1)
    sd = sd_ref[...]
    sdc = jnp.broadcast_to(sd, (_PREP_RB, HALF))
    tfull = tab_ref[...]
    tb = jnp.where(h == 0, tfull[:, :HALF], tfull[:, HALF:])
    node0_ref[...] = (tb * sdc)[None]
    tsplit_ref[...] = tb[None]
    sdb_ref[...] = sdc[None]
    sd2b_ref[...] = (sdc * sdc)[None]


def _prep_call(table, sd2d):
    shp = jax.ShapeDtypeStruct((NC, NPAD, HALF), jnp.float32)
    return pl.pallas_call(
        _prep_body,
        grid=(NPAD // _PREP_RB, NC),
        in_specs=[
            pl.BlockSpec((_PREP_RB, 64), lambda b, h: (b, 0)),
            pl.BlockSpec((_PREP_RB, 1), lambda b, h: (b, 0)),
        ],
        out_specs=[pl.BlockSpec((1, _PREP_RB, HALF), lambda b, h: (h, b, 0))] * 4,
        out_shape=(shp,) * 4,
    )(table, sd2d)


_EW_BLK = ROWS_PER_TILE


def _scale_body(y_ref, sd2b_ref, o_ref):
    o_ref[...] = y_ref[...] * sd2b_ref[...]


def _scale_call(y, sd2b):
    spec = pl.BlockSpec((NC, _EW_BLK, HALF), lambda b: (0, b, 0))
    return pl.pallas_call(
        _scale_body,
        grid=(NPAD // _EW_BLK,),
        in_specs=[spec, spec],
        out_specs=spec,
        out_shape=jax.ShapeDtypeStruct((NC, NPAD, HALF), jnp.float32),
    )(y, sd2b)


def _propfinal_body(t_ref, y1_ref, y2_ref, y3_ref, sdb_ref, o_ref):
    ysum = y1_ref[...] + y2_ref[...] + y3_ref[...]
    o_ref[...] = 0.25 * (t_ref[...] + ysum * sdb_ref[...])


def _propfinal_call(tsplit, y1, y2, y3, sdb):
    spec = pl.BlockSpec((NC, _EW_BLK, HALF), lambda b: (0, b, 0))
    return pl.pallas_call(
        _propfinal_body,
        grid=(NPAD // _EW_BLK,),
        in_specs=[spec] * 5,
        out_specs=spec,
        out_shape=jax.ShapeDtypeStruct((NC, NPAD, HALF), jnp.float32),
    )(tsplit, y1, y2, y3, sdb)


def _loss_body(ego_ref, pg_ref, out_ref):
    e = ego_ref[...]
    reg = jnp.sum(e * e)
    pg = pg_ref[...]
    pos_s = jnp.sum(pg[0] * pg[2] + pg[1] * pg[3], axis=1)
    neg_s = jnp.sum(pg[0] * pg[4] + pg[1] * pg[5], axis=1)
    x = neg_s - pos_s
    sp = jnp.maximum(x, 0.0) + jnp.log1p(jnp.exp(-jnp.abs(x)))
    loss = jnp.mean(sp) + LAM * 0.5 * reg / float(BATCH)
    out_ref[...] = jnp.full((8, 128), loss, jnp.float32)


def _loss_call(ego_g, prop_g):
    return pl.pallas_call(
        _loss_body,
        out_shape=jax.ShapeDtypeStruct((8, 128), jnp.float32),
    )(ego_g, prop_g)


def kernel(embedding_table, edge_index, users, pos, neg):
    src = edge_index[0]
    dst = edge_index[1]
    npad = EPAD - N_EDGES
    srcp = jnp.concatenate([src, jnp.zeros((npad,), jnp.int32)])
    dstp = jnp.concatenate([dst, jnp.full((npad,), TRASH, jnp.int32)])
    srcs = jnp.stack([srcp, srcp + NPAD]).reshape(NC, NS, LAYER_CHUNKS, CHUNK)
    dsts_layer = dstp.reshape(NS, LAYER_CHUNKS, CHUNK)
    dsts_deg = dstp.reshape(NW, DEG_CHUNKS, DEG_CHUNK)

    ones_rows = jnp.ones((DEG_CHUNK, 16), jnp.float32)
    zeros_deg = jnp.zeros((ROWS_PER_TILE, 16), jnp.float32)
    deg_parts = _deg_call(dsts_deg, ones_rows, zeros_deg)
    sd2d = _sd_call(deg_parts)
    node0, tsplit, sdb, sd2b = _prep_call(embedding_table, sd2d)

    zeros_tile = jnp.zeros((ROWS_PER_TILE, HALF), jnp.float32)
    node = node0
    ys = []
    for l in range(N_LAYERS):
        y = _layer_call(node.reshape(NC * NPAD, HALF), srcs, dsts_layer,
                        zeros_tile)
        ys.append(y)
        if l < N_LAYERS - 1:
            node = _scale_call(y, sd2b)
    prop = _propfinal_call(tsplit, ys[0], ys[1], ys[2], sdb)

    idx_ego = jnp.stack([users, pos + N_USERS, neg + N_USERS])
    idx_ego = idx_ego.reshape(3, NW, 128)
    u0, p0, n0 = users, pos + N_USERS, neg + N_USERS
    idx_prop = jnp.stack([u0, u0 + NPAD, p0, p0 + NPAD, n0, n0 + NPAD])
    idx_prop = idx_prop.reshape(6, NW, 128)
    ego_g, prop_g = _gather_call(embedding_table,
                                 prop.reshape(NC * NPAD, HALF),
                                 idx_ego, idx_prop)
    out = _loss_call(ego_g, prop_g)
    return out[0, 0]
